# baseline (speedup 1.0000x reference)
import functools

import jax
import jax.numpy as jnp
from jax import lax
from jax.experimental import pallas as pl
from jax.experimental.pallas import tpu as pltpu
from jax.experimental.pallas import tpu_sc as plsc

NC = 2
NS = 16
F = 256
FH = 128


def _sc_mesh():
  return plsc.VectorSubcoreMesh(
      core_axis_name="c", subcore_axis_name="s", num_cores=NC,
      num_subcores=NS)


def _make_deg_kernel(NP, E):
  ew = E // (NC * NS)
  assert ew % 8 == 0

  @functools.partial(
      pl.kernel,
      out_type=jax.ShapeDtypeStruct((NC, NP), jnp.float32),
      mesh=_sc_mesh(),
      scratch_types=[
          pltpu.VMEM_SHARED((NP,), jnp.float32),
          pltpu.VMEM((ew,), jnp.int32),
          pltpu.VMEM((ew,), jnp.float32),
      ],
  )
  def deg_kernel(dst_hbm, ones_hbm, zeros_hbm, out_hbm, acc_sh, idx_v, one_v):
    c = lax.axis_index("c")
    s = lax.axis_index("s")
    wid = c * NS + s

    @pl.when(s == 0)
    def _():
      pltpu.sync_copy(zeros_hbm, acc_sh)
    plsc.subcore_barrier()

    base = wid * ew
    pltpu.sync_copy(dst_hbm.at[pl.ds(base, ew)], idx_v)
    pltpu.sync_copy(ones_hbm, one_v)
    pltpu.sync_copy(one_v, acc_sh.at[idx_v], add=True)
    plsc.subcore_barrier()

    @pl.when(s == 0)
    def _():
      pltpu.sync_copy(acc_sh, out_hbm.at[c])

  return deg_kernel


def _dense1_body(x_ref, w_ref, p_ref, ylo_ref, yhi_ref, dis_ref):
  p = p_ref[0]
  dis = lax.rsqrt(1.0 + p[0, :] + p[1, :])
  xw = jnp.dot(x_ref[...], w_ref[...], preferred_element_type=jnp.float32)
  y = xw * dis[:, None]
  ylo_ref[...] = y[:, :FH]
  yhi_ref[...] = y[:, FH:]
  dis_ref[0, 0, :] = dis


def _dense1(x, W1, partials3, N, B=1000):
  grid = (N // B,)
  return pl.pallas_call(
      _dense1_body,
      grid=grid,
      in_specs=[
          pl.BlockSpec((B, F), lambda i: (i, 0)),
          pl.BlockSpec((F, F), lambda i: (0, 0)),
          pl.BlockSpec((1, NC, B), lambda i: (i, 0, 0)),
      ],
      out_specs=[
          pl.BlockSpec((B, FH), lambda i: (i, 0)),
          pl.BlockSpec((B, FH), lambda i: (i, 0)),
          pl.BlockSpec((1, 1, B), lambda i: (i, 0, 0)),
      ],
      out_shape=[
          jax.ShapeDtypeStruct((N, FH), jnp.float32),
          jax.ShapeDtypeStruct((N, FH), jnp.float32),
          jax.ShapeDtypeStruct((N // B, 1, B), jnp.float32),
      ],
  )(x, W1, partials3)


def _make_agg_kernel(N, E, K, nchunk):
  assert K % 8 == 0

  npair = nchunk // 2
  assert nchunk % 2 == 0

  @functools.partial(
      pl.kernel,
      out_type=[
          jax.ShapeDtypeStruct((N, FH), jnp.float32),
          jax.ShapeDtypeStruct((N, FH), jnp.float32),
      ],
      mesh=_sc_mesh(),
      compiler_params=pltpu.CompilerParams(use_tc_tiling_on_sc=False),
      scratch_types=[
          pltpu.VMEM_SHARED((N + 8, FH), jnp.float32),
          pltpu.VMEM((4, K), jnp.int32),
          pltpu.VMEM((K, FH), jnp.float32),
          pltpu.VMEM((K, FH), jnp.float32),
          pltpu.SemaphoreType.DMA,
          pltpu.SemaphoreType.DMA,
          pltpu.SemaphoreType.DMA,
          pltpu.SemaphoreType.DMA,
      ],
  )
  def agg_kernel(idx_hbm, ylo_hbm, yhi_hbm, olo_hbm, ohi_hbm,
                 acc_sh, idxb, rowsa, rowsb, gsa, gsb, ssa, ssb):
    c = lax.axis_index("c")
    s = lax.axis_index("s")

    NR = N // NS

    def run(y_hbm, o_hbm):
      pltpu.sync_copy(y_hbm.at[pl.ds(s * NR, NR)],
                      acc_sh.at[pl.ds(s * NR, NR)])
      plsc.subcore_barrier()

      def pair(i, _):
        pltpu.sync_copy(idx_hbm.at[s].at[i], idxb)
        ga = pltpu.async_copy(y_hbm.at[idxb.at[0]], rowsa, gsa)
        gb = pltpu.async_copy(y_hbm.at[idxb.at[1]], rowsb, gsb)
        ga.wait()
        sa = pltpu.async_copy(rowsa, acc_sh.at[idxb.at[2]], ssa, add=True)
        gb.wait()
        sb = pltpu.async_copy(rowsb, acc_sh.at[idxb.at[3]], ssb, add=True)
        sa.wait()
        sb.wait()
        return 0
      lax.fori_loop(0, npair, pair, 0)
      plsc.subcore_barrier()
      pltpu.sync_copy(acc_sh.at[pl.ds(s * NR, NR)],
                      o_hbm.at[pl.ds(s * NR, NR)])

    @pl.when(c == 0)
    def _():
      run(ylo_hbm, olo_hbm)

    @pl.when(c == 1)
    def _():
      run(yhi_hbm, ohi_hbm)

  return agg_kernel


def _dense2_body(alo_ref, ahi_ref, dis_ref, b1_ref, w2_ref, z_ref):
  dis = dis_ref[0, 0, :]
  b1 = b1_ref[...]
  w2 = w2_ref[...]
  h_lo = jnp.maximum(alo_ref[...] * dis[:, None] + b1[:FH], 0.0)
  h_hi = jnp.maximum(ahi_ref[...] * dis[:, None] + b1[FH:], 0.0)
  z = (jnp.dot(h_lo, w2[:FH, :], preferred_element_type=jnp.float32)
       + jnp.dot(h_hi, w2[FH:, :], preferred_element_type=jnp.float32))
  z_ref[0, 0, :] = z[:, 0] * dis


def _dense2(acc_lo, acc_hi, dis3, b1, W2, N, B=1000):
  grid = (N // B,)
  return pl.pallas_call(
      _dense2_body,
      grid=grid,
      in_specs=[
          pl.BlockSpec((B, FH), lambda i: (i, 0)),
          pl.BlockSpec((B, FH), lambda i: (i, 0)),
          pl.BlockSpec((1, 1, B), lambda i: (i, 0, 0)),
          pl.BlockSpec((F,), lambda i: (0,)),
          pl.BlockSpec((F, 1), lambda i: (0, 0)),
      ],
      out_specs=pl.BlockSpec((1, 1, B), lambda i: (i, 0, 0)),
      out_shape=jax.ShapeDtypeStruct((N // B, 1, B), jnp.float32),
  )(acc_lo, acc_hi, dis3, b1, W2)


def _make_scal_kernel(NP, E):
  ew = E // (NC * NS)
  assert ew % 8 == 0

  @functools.partial(
      pl.kernel,
      out_type=jax.ShapeDtypeStruct((NC, NP), jnp.float32),
      mesh=_sc_mesh(),
      scratch_types=[
          pltpu.VMEM_SHARED((NP,), jnp.float32),
          pltpu.VMEM((ew,), jnp.int32),
          pltpu.VMEM((ew,), jnp.int32),
          pltpu.VMEM((ew,), jnp.float32),
      ],
  )
  def scal_kernel(src_hbm, dst_hbm, z_hbm, zeros_hbm, out_hbm,
                  acc_sh, srcv, dstv, valv):
    c = lax.axis_index("c")
    s = lax.axis_index("s")
    wid = c * NS + s

    @pl.when((s == 0) & (c == 0))
    def _():
      pltpu.sync_copy(z_hbm, acc_sh)

    @pl.when((s == 0) & (c == 1))
    def _():
      pltpu.sync_copy(zeros_hbm, acc_sh)
    plsc.subcore_barrier()

    base = wid * ew
    pltpu.sync_copy(src_hbm.at[pl.ds(base, ew)], srcv)
    pltpu.sync_copy(dst_hbm.at[pl.ds(base, ew)], dstv)
    pltpu.sync_copy(z_hbm.at[srcv], valv)
    pltpu.sync_copy(valv, acc_sh.at[dstv], add=True)
    plsc.subcore_barrier()

    @pl.when(s == 0)
    def _():
      pltpu.sync_copy(acc_sh, out_hbm.at[c])

  return scal_kernel


def _final_body(p_ref, dis_ref, b2_ref, o_ref):
  p = p_ref[...]
  o_ref[...] = dis_ref[...] * (p[0, :] + p[1, :]) + b2_ref[0]


def _final(partials, dis, b2, N):
  return pl.pallas_call(
      _final_body,
      out_shape=jax.ShapeDtypeStruct((N,), jnp.float32),
  )(partials, dis, b2)


def kernel(x, edge_index, W1, b1, W2, b2):
  N, _ = x.shape
  E = edge_index.shape[1]
  NP = ((N + 127) // 128) * 128
  src = edge_index[0].astype(jnp.int32)
  dst = edge_index[1].astype(jnp.int32)
  ew = E // (NC * NS)
  ones_e = jnp.ones((ew,), jnp.float32)
  zeros_np = jnp.zeros((NP,), jnp.float32)

  B = 1000
  deg_partials = _make_deg_kernel(NP, E)(dst, ones_e, zeros_np)[:, :N]
  partials3 = deg_partials.reshape(NC, N // B, B).transpose(1, 0, 2)
  y_lo, y_hi, dis3 = _dense1(x, W1, partials3, N, B)
  K = 192
  ew = E // NS
  nchunk = -(-ew // K)
  if nchunk % 2:
    nchunk += 1
  pad = nchunk * K - ew
  src2 = src.reshape(NS, ew)
  dst2 = dst.reshape(NS, ew)
  if pad:
    pad_src = jnp.zeros((NS, pad), jnp.int32)
    pad_dst = jnp.broadcast_to(
        N + (jnp.arange(pad, dtype=jnp.int32) % 8), (NS, pad))
    src2 = jnp.concatenate([src2, pad_src], axis=1)
    dst2 = jnp.concatenate([dst2, pad_dst], axis=1)
  srcr = src2.reshape(NS, nchunk // 2, 2, K)
  dstr = dst2.reshape(NS, nchunk // 2, 2, K)
  idx4 = jnp.concatenate([srcr, dstr], axis=2)
  acc_lo, acc_hi = _make_agg_kernel(N, E, K, nchunk)(idx4, y_lo, y_hi)
  z3 = _dense2(acc_lo, acc_hi, dis3, b1, W2, N, B)
  z_pad = jnp.concatenate([z3.reshape(N), jnp.zeros((NP - N,), jnp.float32)])
  partials = _make_scal_kernel(NP, E)(src, dst, z_pad, zeros_np)[:, :N]
  out = _final(partials, dis3.reshape(N), b2, N)
  return out


# speedup vs baseline: 1.9011x; 1.9011x over previous
import functools

import jax
import jax.numpy as jnp
from jax import lax
from jax.experimental import pallas as pl
from jax.experimental.pallas import tpu as pltpu
from jax.experimental.pallas import tpu_sc as plsc

NC = 2
NS = 16
F = 256
FH = 128


def _sc_mesh():
  return plsc.VectorSubcoreMesh(
      core_axis_name="c", subcore_axis_name="s", num_cores=NC,
      num_subcores=NS)


def _make_deg_kernel(NP, E):
  ew = E // (NC * NS)
  assert ew % 8 == 0

  @functools.partial(
      pl.kernel,
      out_type=jax.ShapeDtypeStruct((NC, NP), jnp.float32),
      mesh=_sc_mesh(),
      scratch_types=[
          pltpu.VMEM_SHARED((NP,), jnp.float32),
          pltpu.VMEM((ew,), jnp.int32),
          pltpu.VMEM((ew,), jnp.float32),
      ],
  )
  def deg_kernel(dst_hbm, ones_hbm, zeros_hbm, out_hbm, acc_sh, idx_v, one_v):
    c = lax.axis_index("c")
    s = lax.axis_index("s")
    wid = c * NS + s

    @pl.when(s == 0)
    def _():
      pltpu.sync_copy(zeros_hbm, acc_sh)
    plsc.subcore_barrier()

    base = wid * ew
    pltpu.sync_copy(dst_hbm.at[pl.ds(base, ew)], idx_v)
    pltpu.sync_copy(ones_hbm, one_v)
    pltpu.sync_copy(one_v, acc_sh.at[idx_v], add=True)
    plsc.subcore_barrier()

    @pl.when(s == 0)
    def _():
      pltpu.sync_copy(acc_sh, out_hbm.at[c])

  return deg_kernel


def _dense1_body(x_ref, w_ref, p_ref, ylo_ref, yhi_ref, dis_ref):
  p = p_ref[0]
  dis = lax.rsqrt(1.0 + p[0, :] + p[1, :])
  xw = jnp.dot(x_ref[...], w_ref[...], preferred_element_type=jnp.float32)
  y = xw * dis[:, None]
  ylo_ref[...] = y[:, :FH]
  yhi_ref[...] = y[:, FH:]
  dis_ref[0, 0, :] = dis


def _dense1(x, W1, partials3, N, B=1000):
  grid = (N // B,)
  return pl.pallas_call(
      _dense1_body,
      grid=grid,
      in_specs=[
          pl.BlockSpec((B, F), lambda i: (i, 0)),
          pl.BlockSpec((F, F), lambda i: (0, 0)),
          pl.BlockSpec((1, NC, B), lambda i: (i, 0, 0)),
      ],
      out_specs=[
          pl.BlockSpec((B, FH), lambda i: (i, 0)),
          pl.BlockSpec((B, FH), lambda i: (i, 0)),
          pl.BlockSpec((1, 1, B), lambda i: (i, 0, 0)),
      ],
      out_shape=[
          jax.ShapeDtypeStruct((N, FH), jnp.float32),
          jax.ShapeDtypeStruct((N, FH), jnp.float32),
          jax.ShapeDtypeStruct((N // B, 1, B), jnp.float32),
      ],
  )(x, W1, partials3)


def _make_agg_kernel(N, E, K, nchunk):
  assert K % 8 == 0

  @functools.partial(
      pl.kernel,
      out_type=[
          jax.ShapeDtypeStruct((N, FH), jnp.float32),
          jax.ShapeDtypeStruct((N, FH), jnp.float32),
      ],
      mesh=_sc_mesh(),
      compiler_params=pltpu.CompilerParams(use_tc_tiling_on_sc=False),
      scratch_types=[
          pltpu.VMEM_SHARED((N, FH), jnp.float32),
          pltpu.VMEM((nchunk, K), jnp.int32),
          pltpu.VMEM((nchunk, K), jnp.int32),
          pltpu.VMEM((K, FH), jnp.float32),
      ],
  )
  def agg_kernel(src_hbm, dst_hbm, ylo_hbm, yhi_hbm, olo_hbm, ohi_hbm,
                 acc_sh, srcv, dstv, rows):
    c = lax.axis_index("c")
    s = lax.axis_index("s")

    pltpu.sync_copy(src_hbm.at[s], srcv)
    pltpu.sync_copy(dst_hbm.at[s], dstv)

    NR = N // NS

    def run(y_hbm, o_hbm):
      pltpu.sync_copy(y_hbm.at[pl.ds(s * NR, NR)],
                      acc_sh.at[pl.ds(s * NR, NR)])
      plsc.subcore_barrier()

      def chunk(j, _):
        pltpu.sync_copy(y_hbm.at[srcv.at[j]], rows)
        pltpu.sync_copy(rows, acc_sh.at[dstv.at[j]], add=True)
        return 0
      lax.fori_loop(0, nchunk, chunk, 0)
      plsc.subcore_barrier()
      pltpu.sync_copy(acc_sh.at[pl.ds(s * NR, NR)],
                      o_hbm.at[pl.ds(s * NR, NR)])

    @pl.when(c == 0)
    def _():
      run(ylo_hbm, olo_hbm)

    @pl.when(c == 1)
    def _():
      run(yhi_hbm, ohi_hbm)

  return agg_kernel


def _dense2_body(alo_ref, ahi_ref, dis_ref, b1_ref, w2_ref, z_ref):
  dis = dis_ref[0, 0, :]
  b1 = b1_ref[...]
  w2 = w2_ref[...]
  h_lo = jnp.maximum(alo_ref[...] * dis[:, None] + b1[:FH], 0.0)
  h_hi = jnp.maximum(ahi_ref[...] * dis[:, None] + b1[FH:], 0.0)
  z = (jnp.dot(h_lo, w2[:FH, :], preferred_element_type=jnp.float32)
       + jnp.dot(h_hi, w2[FH:, :], preferred_element_type=jnp.float32))
  z_ref[0, 0, :] = z[:, 0] * dis


def _dense2(acc_lo, acc_hi, dis3, b1, W2, N, B=1000):
  grid = (N // B,)
  return pl.pallas_call(
      _dense2_body,
      grid=grid,
      in_specs=[
          pl.BlockSpec((B, FH), lambda i: (i, 0)),
          pl.BlockSpec((B, FH), lambda i: (i, 0)),
          pl.BlockSpec((1, 1, B), lambda i: (i, 0, 0)),
          pl.BlockSpec((F,), lambda i: (0,)),
          pl.BlockSpec((F, 1), lambda i: (0, 0)),
      ],
      out_specs=pl.BlockSpec((1, 1, B), lambda i: (i, 0, 0)),
      out_shape=jax.ShapeDtypeStruct((N // B, 1, B), jnp.float32),
  )(acc_lo, acc_hi, dis3, b1, W2)


def _make_scal_kernel(NP, E):
  ew = E // (NC * NS)
  assert ew % 8 == 0

  @functools.partial(
      pl.kernel,
      out_type=jax.ShapeDtypeStruct((NC, NP), jnp.float32),
      mesh=_sc_mesh(),
      scratch_types=[
          pltpu.VMEM_SHARED((NP,), jnp.float32),
          pltpu.VMEM((ew,), jnp.int32),
          pltpu.VMEM((ew,), jnp.int32),
          pltpu.VMEM((ew,), jnp.float32),
      ],
  )
  def scal_kernel(src_hbm, dst_hbm, z_hbm, zeros_hbm, out_hbm,
                  acc_sh, srcv, dstv, valv):
    c = lax.axis_index("c")
    s = lax.axis_index("s")
    wid = c * NS + s

    @pl.when((s == 0) & (c == 0))
    def _():
      pltpu.sync_copy(z_hbm, acc_sh)

    @pl.when((s == 0) & (c == 1))
    def _():
      pltpu.sync_copy(zeros_hbm, acc_sh)
    plsc.subcore_barrier()

    base = wid * ew
    pltpu.sync_copy(src_hbm.at[pl.ds(base, ew)], srcv)
    pltpu.sync_copy(dst_hbm.at[pl.ds(base, ew)], dstv)
    pltpu.sync_copy(z_hbm.at[srcv], valv)
    pltpu.sync_copy(valv, acc_sh.at[dstv], add=True)
    plsc.subcore_barrier()

    @pl.when(s == 0)
    def _():
      pltpu.sync_copy(acc_sh, out_hbm.at[c])

  return scal_kernel


def _final_body(p_ref, dis_ref, b2_ref, o_ref):
  p = p_ref[...]
  o_ref[...] = dis_ref[...] * (p[0, :] + p[1, :]) + b2_ref[0]


def _final(partials, dis, b2, N):
  return pl.pallas_call(
      _final_body,
      out_shape=jax.ShapeDtypeStruct((N,), jnp.float32),
  )(partials, dis, b2)


def kernel(x, edge_index, W1, b1, W2, b2):
  N, _ = x.shape
  E = edge_index.shape[1]
  NP = ((N + 127) // 128) * 128
  src = edge_index[0].astype(jnp.int32)
  dst = edge_index[1].astype(jnp.int32)
  ew = E // (NC * NS)
  ones_e = jnp.ones((ew,), jnp.float32)
  zeros_np = jnp.zeros((NP,), jnp.float32)

  B = 1000
  deg_partials = _make_deg_kernel(NP, E)(dst, ones_e, zeros_np)[:, :N]
  partials3 = deg_partials.reshape(NC, N // B, B).transpose(1, 0, 2)
  y_lo, y_hi, dis3 = _dense1(x, W1, partials3, N, B)
  K = 200
  ew = E // NS
  nchunk = ew // K
  assert ew % K == 0
  src3 = src.reshape(NS, nchunk, K)
  dst3 = dst.reshape(NS, nchunk, K)
  acc_lo, acc_hi = _make_agg_kernel(N, E, K, nchunk)(src3, dst3, y_lo, y_hi)
  z3 = _dense2(acc_lo, acc_hi, dis3, b1, W2, N, B)
  z_pad = jnp.concatenate([z3.reshape(N), jnp.zeros((NP - N,), jnp.float32)])
  partials = _make_scal_kernel(NP, E)(src, dst, z_pad, zeros_np)[:, :N]
  out = _final(partials, dis3.reshape(N), b2, N)
  return out


# speedup vs baseline: 2.0536x; 1.0802x over previous
import functools

import jax
import jax.numpy as jnp
from jax import lax
from jax.experimental import pallas as pl
from jax.experimental.pallas import tpu as pltpu
from jax.experimental.pallas import tpu_sc as plsc

NC = 2
NS = 16
F = 256
FH = 128


def _sc_mesh():
  return plsc.VectorSubcoreMesh(
      core_axis_name="c", subcore_axis_name="s", num_cores=NC,
      num_subcores=NS)


def _make_deg_kernel(NP, E):
  ew = E // (NC * NS)
  assert ew % 8 == 0

  @functools.partial(
      pl.kernel,
      out_type=jax.ShapeDtypeStruct((NC, NP), jnp.float32),
      mesh=_sc_mesh(),
      scratch_types=[
          pltpu.VMEM_SHARED((NP,), jnp.float32),
          pltpu.VMEM((ew,), jnp.int32),
          pltpu.VMEM((ew,), jnp.float32),
      ],
  )
  def deg_kernel(dst_hbm, ones_hbm, zeros_hbm, out_hbm, acc_sh, idx_v, one_v):
    c = lax.axis_index("c")
    s = lax.axis_index("s")
    wid = c * NS + s

    @pl.when(s == 0)
    def _():
      pltpu.sync_copy(zeros_hbm, acc_sh)
    plsc.subcore_barrier()

    base = wid * ew
    pltpu.sync_copy(dst_hbm.at[pl.ds(base, ew)], idx_v)
    pltpu.sync_copy(ones_hbm, one_v)
    pltpu.sync_copy(one_v, acc_sh.at[idx_v], add=True)
    plsc.subcore_barrier()

    @pl.when(s == 0)
    def _():
      pltpu.sync_copy(acc_sh, out_hbm.at[c])

  return deg_kernel


def _dense1_body(x_ref, w_ref, p_ref, ylo_ref, yhi_ref, dis_ref):
  p = p_ref[0]
  dis = lax.rsqrt(1.0 + p[0, :] + p[1, :])
  xw = jnp.dot(x_ref[...], w_ref[...], preferred_element_type=jnp.float32)
  y = xw * dis[:, None]
  ylo_ref[...] = y[:, :FH]
  yhi_ref[...] = y[:, FH:]
  dis_ref[0, 0, :] = dis


def _dense1(x, W1, partials3, N, B=1000):
  grid = (N // B,)
  return pl.pallas_call(
      _dense1_body,
      grid=grid,
      in_specs=[
          pl.BlockSpec((B, F), lambda i: (i, 0)),
          pl.BlockSpec((F, F), lambda i: (0, 0)),
          pl.BlockSpec((1, NC, B), lambda i: (i, 0, 0)),
      ],
      out_specs=[
          pl.BlockSpec((B, FH), lambda i: (i, 0)),
          pl.BlockSpec((B, FH), lambda i: (i, 0)),
          pl.BlockSpec((1, 1, B), lambda i: (i, 0, 0)),
      ],
      out_shape=[
          jax.ShapeDtypeStruct((N, FH), jnp.float32),
          jax.ShapeDtypeStruct((N, FH), jnp.float32),
          jax.ShapeDtypeStruct((N // B, 1, B), jnp.float32),
      ],
  )(x, W1, partials3)


def _make_agg_kernel(N, E, K, nchunk):
  assert K % 8 == 0

  @functools.partial(
      pl.kernel,
      out_type=[
          jax.ShapeDtypeStruct((N, FH), jnp.float32),
          jax.ShapeDtypeStruct((N, FH), jnp.float32),
      ],
      mesh=_sc_mesh(),
      compiler_params=pltpu.CompilerParams(use_tc_tiling_on_sc=False),
      scratch_types=[
          pltpu.VMEM_SHARED((N, FH), jnp.float32),
          pltpu.VMEM((nchunk, K), jnp.int32),
          pltpu.VMEM((nchunk, K), jnp.int32),
          pltpu.VMEM((K, FH), jnp.float32),
      ],
  )
  def agg_kernel(src_hbm, dst_hbm, ylo_hbm, yhi_hbm, olo_hbm, ohi_hbm,
                 acc_sh, srcv, dstv, rows):
    c = lax.axis_index("c")
    s = lax.axis_index("s")

    pltpu.sync_copy(src_hbm.at[s], srcv)
    pltpu.sync_copy(dst_hbm.at[s], dstv)

    NR = N // NS

    def run(y_hbm, o_hbm):
      pltpu.sync_copy(y_hbm.at[pl.ds(s * NR, NR)],
                      acc_sh.at[pl.ds(s * NR, NR)])
      plsc.subcore_barrier()

      def chunk(j, _):
        pltpu.sync_copy(y_hbm.at[srcv.at[j]], rows)
        pltpu.sync_copy(rows, acc_sh.at[dstv.at[j]], add=True)
        return 0
      lax.fori_loop(0, nchunk, chunk, 0)
      plsc.subcore_barrier()
      pltpu.sync_copy(acc_sh.at[pl.ds(s * NR, NR)],
                      o_hbm.at[pl.ds(s * NR, NR)])

    @pl.when(c == 0)
    def _():
      run(ylo_hbm, olo_hbm)

    @pl.when(c == 1)
    def _():
      run(yhi_hbm, ohi_hbm)

  return agg_kernel


def _dense2_body(alo_ref, ahi_ref, dis_ref, b1_ref, w2_ref, z_ref):
  dis = dis_ref[0, 0, :]
  b1 = b1_ref[...]
  w2 = w2_ref[...]
  h_lo = jnp.maximum(alo_ref[...] * dis[:, None] + b1[:FH], 0.0)
  h_hi = jnp.maximum(ahi_ref[...] * dis[:, None] + b1[FH:], 0.0)
  z = (jnp.dot(h_lo, w2[:FH, :], preferred_element_type=jnp.float32)
       + jnp.dot(h_hi, w2[FH:, :], preferred_element_type=jnp.float32))
  z_ref[0, 0, :] = z[:, 0] * dis


def _dense2(acc_lo, acc_hi, dis3, b1, W2, N, B=1000):
  grid = (N // B,)
  return pl.pallas_call(
      _dense2_body,
      grid=grid,
      in_specs=[
          pl.BlockSpec((B, FH), lambda i: (i, 0)),
          pl.BlockSpec((B, FH), lambda i: (i, 0)),
          pl.BlockSpec((1, 1, B), lambda i: (i, 0, 0)),
          pl.BlockSpec((F,), lambda i: (0,)),
          pl.BlockSpec((F, 1), lambda i: (0, 0)),
      ],
      out_specs=pl.BlockSpec((1, 1, B), lambda i: (i, 0, 0)),
      out_shape=jax.ShapeDtypeStruct((N // B, 1, B), jnp.float32),
  )(acc_lo, acc_hi, dis3, b1, W2)


def _make_scal_kernel(NP, E):
  ew = E // (NC * NS)
  ewp = ((ew + 15) // 16) * 16
  assert ew % 8 == 0

  @functools.partial(
      pl.kernel,
      out_type=jax.ShapeDtypeStruct((NC, NP), jnp.float32),
      mesh=_sc_mesh(),
      compiler_params=pltpu.CompilerParams(needs_layout_passes=False),
      scratch_types=[
          pltpu.VMEM_SHARED((NP,), jnp.float32),
          pltpu.VMEM((NP,), jnp.float32),
          pltpu.VMEM((ewp,), jnp.int32),
          pltpu.VMEM((ew,), jnp.int32),
          pltpu.VMEM((ewp,), jnp.float32),
      ],
  )
  def scal_kernel(src_hbm, dst_hbm, z_hbm, zeros_hbm, out_hbm,
                  acc_sh, zloc, srcv, dstv, valv):
    c = lax.axis_index("c")
    s = lax.axis_index("s")
    wid = c * NS + s

    @pl.when((s == 0) & (c == 0))
    def _():
      pltpu.sync_copy(z_hbm, acc_sh)

    @pl.when((s == 0) & (c == 1))
    def _():
      pltpu.sync_copy(zeros_hbm, acc_sh)

    base = wid * ew
    pltpu.sync_copy(src_hbm.at[pl.ds(base, ew)], srcv.at[pl.ds(0, ew)])
    pltpu.sync_copy(dst_hbm.at[pl.ds(base, ew)], dstv)
    pltpu.sync_copy(z_hbm, zloc)

    nfull = ew // 16

    def gath(i, _):
      idx = srcv[pl.ds(i * 16, 16)]
      valv[pl.ds(i * 16, 16)] = plsc.load_gather(zloc, [idx])
      return 0
    lax.fori_loop(0, nfull, gath, 0)
    rem = ew - nfull * 16
    if rem:
      lane = lax.iota(jnp.int32, 16)
      m = lane < rem
      idx = jnp.where(m, srcv[pl.ds(nfull * 16, 16)], 0)
      valv[pl.ds(nfull * 16, 16)] = plsc.load_gather(zloc, [idx])

    plsc.subcore_barrier()
    pltpu.sync_copy(valv.at[pl.ds(0, ew)], acc_sh.at[dstv], add=True)
    plsc.subcore_barrier()

    @pl.when(s == 0)
    def _():
      pltpu.sync_copy(acc_sh, out_hbm.at[c])

  return scal_kernel


def _final_body(p_ref, dis_ref, b2_ref, o_ref):
  p = p_ref[...]
  o_ref[...] = dis_ref[...] * (p[0, :] + p[1, :]) + b2_ref[0]


def _final(partials, dis, b2, N):
  return pl.pallas_call(
      _final_body,
      out_shape=jax.ShapeDtypeStruct((N,), jnp.float32),
  )(partials, dis, b2)


def kernel(x, edge_index, W1, b1, W2, b2):
  N, _ = x.shape
  E = edge_index.shape[1]
  NP = ((N + 127) // 128) * 128
  src = edge_index[0].astype(jnp.int32)
  dst = edge_index[1].astype(jnp.int32)
  ew = E // (NC * NS)
  ones_e = jnp.ones((ew,), jnp.float32)
  zeros_np = jnp.zeros((NP,), jnp.float32)

  B = 1000
  deg_partials = _make_deg_kernel(NP, E)(dst, ones_e, zeros_np)[:, :N]
  partials3 = deg_partials.reshape(NC, N // B, B).transpose(1, 0, 2)
  y_lo, y_hi, dis3 = _dense1(x, W1, partials3, N, B)
  K = 200
  ew = E // NS
  nchunk = ew // K
  assert ew % K == 0
  src3 = src.reshape(NS, nchunk, K)
  dst3 = dst.reshape(NS, nchunk, K)
  acc_lo, acc_hi = _make_agg_kernel(N, E, K, nchunk)(src3, dst3, y_lo, y_hi)
  z3 = _dense2(acc_lo, acc_hi, dis3, b1, W2, N, B)
  z_pad = jnp.concatenate([z3.reshape(N), jnp.zeros((NP - N,), jnp.float32)])
  partials = _make_scal_kernel(NP, E)(src, dst, z_pad, zeros_np)[:, :N]
  out = _final(partials, dis3.reshape(N), b2, N)
  return out


# speedup vs baseline: 2.0590x; 1.0026x over previous
import functools

import jax
import jax.numpy as jnp
from jax import lax
from jax.experimental import pallas as pl
from jax.experimental.pallas import tpu as pltpu
from jax.experimental.pallas import tpu_sc as plsc

NC = 2
NS = 16
F = 256
FH = 128


def _sc_mesh():
  return plsc.VectorSubcoreMesh(
      core_axis_name="c", subcore_axis_name="s", num_cores=NC,
      num_subcores=NS)


def _make_deg_kernel(NP, E):
  ew = E // (NC * NS)
  assert ew % 8 == 0

  @functools.partial(
      pl.kernel,
      out_type=jax.ShapeDtypeStruct((NC, NP), jnp.float32),
      mesh=_sc_mesh(),
      scratch_types=[
          pltpu.VMEM_SHARED((NP,), jnp.float32),
          pltpu.VMEM((ew,), jnp.int32),
          pltpu.VMEM((ew,), jnp.float32),
      ],
  )
  def deg_kernel(dst_hbm, ones_hbm, zeros_hbm, out_hbm, acc_sh, idx_v, one_v):
    c = lax.axis_index("c")
    s = lax.axis_index("s")
    wid = c * NS + s

    @pl.when(s == 0)
    def _():
      pltpu.sync_copy(zeros_hbm, acc_sh)
    plsc.subcore_barrier()

    base = wid * ew
    pltpu.sync_copy(dst_hbm.at[pl.ds(base, ew)], idx_v)
    pltpu.sync_copy(ones_hbm, one_v)
    pltpu.sync_copy(one_v, acc_sh.at[idx_v], add=True)
    plsc.subcore_barrier()

    @pl.when(s == 0)
    def _():
      pltpu.sync_copy(acc_sh, out_hbm.at[c])

  return deg_kernel


def _dense1_body(x_ref, w_ref, p_ref, ylo_ref, yhi_ref, dis_ref):
  p = p_ref[0]
  dis = lax.rsqrt(1.0 + p[0, :] + p[1, :])
  xw = jnp.dot(x_ref[...], w_ref[...], preferred_element_type=jnp.float32)
  y = xw * dis[:, None]
  ylo_ref[...] = y[:, :FH]
  yhi_ref[...] = y[:, FH:]
  dis_ref[0, 0, :] = dis


def _dense1(x, W1, partials3, N, B=1000):
  grid = (N // B,)
  return pl.pallas_call(
      _dense1_body,
      grid=grid,
      in_specs=[
          pl.BlockSpec((B, F), lambda i: (i, 0)),
          pl.BlockSpec((F, F), lambda i: (0, 0)),
          pl.BlockSpec((1, NC, B), lambda i: (i, 0, 0)),
      ],
      out_specs=[
          pl.BlockSpec((B, FH), lambda i: (i, 0)),
          pl.BlockSpec((B, FH), lambda i: (i, 0)),
          pl.BlockSpec((1, 1, B), lambda i: (i, 0, 0)),
      ],
      out_shape=[
          jax.ShapeDtypeStruct((N, FH), jnp.float32),
          jax.ShapeDtypeStruct((N, FH), jnp.float32),
          jax.ShapeDtypeStruct((N // B, 1, B), jnp.float32),
      ],
  )(x, W1, partials3)


def _make_agg_kernel(N, E, K, nchunk):
  assert K % 8 == 0

  npair = nchunk // 2
  assert nchunk % 2 == 0

  @functools.partial(
      pl.kernel,
      out_type=[
          jax.ShapeDtypeStruct((N, FH), jnp.float32),
          jax.ShapeDtypeStruct((N, FH), jnp.float32),
      ],
      mesh=_sc_mesh(),
      compiler_params=pltpu.CompilerParams(use_tc_tiling_on_sc=False),
      scratch_types=[
          pltpu.VMEM_SHARED((N + 128, FH), jnp.float32),
          pltpu.VMEM((4, K), jnp.int32),
          pltpu.VMEM((K, FH), jnp.float32),
          pltpu.VMEM((K, FH), jnp.float32),
          pltpu.SemaphoreType.DMA,
          pltpu.SemaphoreType.DMA,
      ],
  )
  def agg_kernel(idx_hbm, ylo_hbm, yhi_hbm, olo_hbm, ohi_hbm,
                 acc_sh, idxb, rowsa, rowsb, ssa, ssb):
    c = lax.axis_index("c")
    s = lax.axis_index("s")

    NR = N // NS

    def run(y_hbm, o_hbm):
      pltpu.sync_copy(y_hbm.at[pl.ds(s * NR, NR)],
                      acc_sh.at[pl.ds(s * NR, NR)])
      plsc.subcore_barrier()

      def pair(i, _):
        pltpu.sync_copy(idx_hbm.at[s].at[i], idxb)
        pltpu.sync_copy(y_hbm.at[idxb.at[0]], rowsa)
        sa = pltpu.async_copy(rowsa, acc_sh.at[idxb.at[2]], ssa, add=True)
        pltpu.sync_copy(y_hbm.at[idxb.at[1]], rowsb)
        sa.wait()
        sb = pltpu.async_copy(rowsb, acc_sh.at[idxb.at[3]], ssb, add=True)
        sb.wait()
        return 0
      lax.fori_loop(0, npair, pair, 0)
      plsc.subcore_barrier()
      pltpu.sync_copy(acc_sh.at[pl.ds(s * NR, NR)],
                      o_hbm.at[pl.ds(s * NR, NR)])

    @pl.when(c == 0)
    def _():
      run(ylo_hbm, olo_hbm)

    @pl.when(c == 1)
    def _():
      run(yhi_hbm, ohi_hbm)

  return agg_kernel


def _dense2_body(alo_ref, ahi_ref, dis_ref, b1_ref, w2_ref, z_ref):
  dis = dis_ref[0, 0, :]
  b1 = b1_ref[...]
  w2 = w2_ref[...]
  h_lo = jnp.maximum(alo_ref[...] * dis[:, None] + b1[:FH], 0.0)
  h_hi = jnp.maximum(ahi_ref[...] * dis[:, None] + b1[FH:], 0.0)
  z = (jnp.dot(h_lo, w2[:FH, :], preferred_element_type=jnp.float32)
       + jnp.dot(h_hi, w2[FH:, :], preferred_element_type=jnp.float32))
  z_ref[0, 0, :] = z[:, 0] * dis


def _dense2(acc_lo, acc_hi, dis3, b1, W2, N, B=1000):
  grid = (N // B,)
  return pl.pallas_call(
      _dense2_body,
      grid=grid,
      in_specs=[
          pl.BlockSpec((B, FH), lambda i: (i, 0)),
          pl.BlockSpec((B, FH), lambda i: (i, 0)),
          pl.BlockSpec((1, 1, B), lambda i: (i, 0, 0)),
          pl.BlockSpec((F,), lambda i: (0,)),
          pl.BlockSpec((F, 1), lambda i: (0, 0)),
      ],
      out_specs=pl.BlockSpec((1, 1, B), lambda i: (i, 0, 0)),
      out_shape=jax.ShapeDtypeStruct((N // B, 1, B), jnp.float32),
  )(acc_lo, acc_hi, dis3, b1, W2)


def _make_scal_kernel(NP, E):
  ew = E // (NC * NS)
  ewp = ((ew + 15) // 16) * 16
  assert ew % 8 == 0

  @functools.partial(
      pl.kernel,
      out_type=jax.ShapeDtypeStruct((NC, NP), jnp.float32),
      mesh=_sc_mesh(),
      compiler_params=pltpu.CompilerParams(needs_layout_passes=False),
      scratch_types=[
          pltpu.VMEM_SHARED((NP,), jnp.float32),
          pltpu.VMEM((NP,), jnp.float32),
          pltpu.VMEM((ewp,), jnp.int32),
          pltpu.VMEM((ew,), jnp.int32),
          pltpu.VMEM((ewp,), jnp.float32),
      ],
  )
  def scal_kernel(src_hbm, dst_hbm, z_hbm, zeros_hbm, out_hbm,
                  acc_sh, zloc, srcv, dstv, valv):
    c = lax.axis_index("c")
    s = lax.axis_index("s")
    wid = c * NS + s

    @pl.when((s == 0) & (c == 0))
    def _():
      pltpu.sync_copy(z_hbm, acc_sh)

    @pl.when((s == 0) & (c == 1))
    def _():
      pltpu.sync_copy(zeros_hbm, acc_sh)

    base = wid * ew
    pltpu.sync_copy(src_hbm.at[pl.ds(base, ew)], srcv.at[pl.ds(0, ew)])
    pltpu.sync_copy(dst_hbm.at[pl.ds(base, ew)], dstv)
    pltpu.sync_copy(z_hbm, zloc)

    nfull = ew // 16

    def gath(i, _):
      idx = srcv[pl.ds(i * 16, 16)]
      valv[pl.ds(i * 16, 16)] = plsc.load_gather(zloc, [idx])
      return 0
    lax.fori_loop(0, nfull, gath, 0)
    rem = ew - nfull * 16
    if rem:
      lane = lax.iota(jnp.int32, 16)
      m = lane < rem
      idx = jnp.where(m, srcv[pl.ds(nfull * 16, 16)], 0)
      valv[pl.ds(nfull * 16, 16)] = plsc.load_gather(zloc, [idx])

    plsc.subcore_barrier()
    pltpu.sync_copy(valv.at[pl.ds(0, ew)], acc_sh.at[dstv], add=True)
    plsc.subcore_barrier()

    @pl.when(s == 0)
    def _():
      pltpu.sync_copy(acc_sh, out_hbm.at[c])

  return scal_kernel


def _final_body(p_ref, dis_ref, b2_ref, o_ref):
  p = p_ref[...]
  o_ref[...] = dis_ref[...] * (p[0, :] + p[1, :]) + b2_ref[0]


def _final(partials, dis, b2, N):
  return pl.pallas_call(
      _final_body,
      out_shape=jax.ShapeDtypeStruct((N,), jnp.float32),
  )(partials, dis, b2)


def kernel(x, edge_index, W1, b1, W2, b2):
  N, _ = x.shape
  E = edge_index.shape[1]
  NP = ((N + 127) // 128) * 128
  src = edge_index[0].astype(jnp.int32)
  dst = edge_index[1].astype(jnp.int32)
  ew = E // (NC * NS)
  ones_e = jnp.ones((ew,), jnp.float32)
  zeros_np = jnp.zeros((NP,), jnp.float32)

  B = 1000
  deg_partials = _make_deg_kernel(NP, E)(dst, ones_e, zeros_np)[:, :N]
  partials3 = deg_partials.reshape(NC, N // B, B).transpose(1, 0, 2)
  y_lo, y_hi, dis3 = _dense1(x, W1, partials3, N, B)
  K = 192
  ew = E // NS
  nchunk = -(-ew // K)
  if nchunk % 2:
    nchunk += 1
  pad = nchunk * K - ew
  src2 = src.reshape(NS, ew)
  dst2 = dst.reshape(NS, ew)
  if pad:
    spread = jnp.arange(pad, dtype=jnp.int32) % 128
    src2 = jnp.concatenate(
        [src2, jnp.broadcast_to(spread, (NS, pad))], axis=1)
    dst2 = jnp.concatenate(
        [dst2, jnp.broadcast_to(N + spread, (NS, pad))], axis=1)
  srcr = src2.reshape(NS, nchunk // 2, 2, K)
  dstr = dst2.reshape(NS, nchunk // 2, 2, K)
  idx4 = jnp.concatenate([srcr, dstr], axis=2)
  acc_lo, acc_hi = _make_agg_kernel(N, E, K, nchunk)(idx4, y_lo, y_hi)
  z3 = _dense2(acc_lo, acc_hi, dis3, b1, W2, N, B)
  z_pad = jnp.concatenate([z3.reshape(N), jnp.zeros((NP - N,), jnp.float32)])
  partials = _make_scal_kernel(NP, E)(src, dst, z_pad, zeros_np)[:, :N]
  out = _final(partials, dis3.reshape(N), b2, N)
  return out
